# baseline (speedup 1.0000x reference)
import jax
import jax.numpy as jnp
from jax import lax
from jax.experimental import pallas as pl
from jax.experimental.pallas import tpu as pltpu
from jax.experimental.pallas import tpu_sc as plsc

NUM_CLASSES = 100000
CODE_DIM = 64
K = 4
BATCH = 16384

_info = plsc.get_sparse_core_info()
_NC, _NS, _L = _info.num_cores, _info.num_subcores, _info.num_lanes
_NW = _NC * _NS
_PW = BATCH // _NW
_CH = 64
_NCHUNK = _PW // _CH
_NV = CODE_DIM // _L


def _sc_body(codes_hbm, pred_hbm, cents_hbm, out_hbm,
             idx_v, codes_v, cents0, cents1, out_v,
             sem_codes, sem_c0, sem_c1):
    wid = lax.axis_index("s") * _NC + lax.axis_index("c")
    base = wid * _PW

    for c in range(_NCHUNK):
        pltpu.sync_copy(pred_hbm.at[pl.ds(base + c * _CH, _CH)], idx_v.at[c])

    codes_cp = pltpu.async_copy(codes_hbm.at[pl.ds(base, _PW), :], codes_v,
                                sem_codes)

    cent_bufs = (cents0, cents1)
    sems = (sem_c0, sem_c1)
    cps = [None, None]
    cps[0] = pltpu.async_copy(cents_hbm.at[idx_v.at[0]], cents0, sem_c0)

    codes_cp.wait()
    lane_last = lax.iota(jnp.int32, _L) == (_L - 1)

    for c in range(_NCHUNK):
        if c + 1 < _NCHUNK:
            nb = (c + 1) % 2
            cps[nb] = pltpu.async_copy(cents_hbm.at[idx_v.at[c + 1]],
                                       cent_bufs[nb], sems[nb])
        cps[c % 2].wait()
        cbuf = cent_bufs[c % 2]

        @plsc.parallel_loop(0, _CH, 1, unroll=4)
        def _item(i, c=c, cbuf=cbuf):
            row = c * _CH + i
            code = [codes_v[row, pl.ds(v * _L, _L)] for v in range(_NV)]
            s = []
            for k in range(K):
                acc = jnp.abs(code[0] - cbuf[i, k, pl.ds(0, _L)])
                for v in range(1, _NV):
                    t = cbuf[i, k, pl.ds(v * _L, _L)]
                    acc = acc + jnp.abs(code[v] - t)
                s.append(plsc.cumsum(acc))
            m = jnp.minimum(jnp.minimum(s[0], s[1]), jnp.minimum(s[2], s[3]))
            m = m * (1.0 / CODE_DIM)
            pos = jnp.full((_L,), row, jnp.int32)
            plsc.store_scatter(out_v, [pos], m, mask=lane_last)

    pltpu.sync_copy(out_v, out_hbm.at[pl.ds(base, _PW)])


_mesh = plsc.VectorSubcoreMesh(core_axis_name="c", subcore_axis_name="s")

_sc_kernel = pl.kernel(
    _sc_body,
    mesh=_mesh,
    out_type=jax.ShapeDtypeStruct((BATCH,), jnp.float32),
    scratch_types=[
        pltpu.VMEM((_NCHUNK, _CH), jnp.int32),
        pltpu.VMEM((_PW, CODE_DIM), jnp.float32),
        pltpu.VMEM((_CH, K, CODE_DIM), jnp.float32),
        pltpu.VMEM((_CH, K, CODE_DIM), jnp.float32),
        pltpu.VMEM((_PW,), jnp.float32),
        pltpu.SemaphoreType.DMA,
        pltpu.SemaphoreType.DMA,
        pltpu.SemaphoreType.DMA,
    ],
    compiler_params=pltpu.CompilerParams(needs_layout_passes=False,
                                         use_tc_tiling_on_sc=False),
)


def kernel(codes, pred_class, centroids):
    pred = pred_class.astype(jnp.int32)
    return _sc_kernel(codes, pred, centroids)


# speedup vs baseline: 2.2819x; 2.2819x over previous
import jax
import jax.numpy as jnp
from jax import lax
from jax.experimental import pallas as pl
from jax.experimental.pallas import tpu as pltpu
from jax.experimental.pallas import tpu_sc as plsc

NUM_CLASSES = 100000
CODE_DIM = 64
K = 4
BATCH = 16384

_info = plsc.get_sparse_core_info()
_NC, _NS, _L = _info.num_cores, _info.num_subcores, _info.num_lanes
_NW = _NC * _NS
_PW = BATCH // _NW
_CH = 64
_NCHUNK = _PW // _CH
_NV = CODE_DIM // _L
_ROWD = K * CODE_DIM
_CP = CODE_DIM + 1


def _sc_body(codes_hbm, pred_hbm, cents_hbm, out_hbm,
             idx_v, slab_v, codep_v, cents0, cents1, out_v,
             sem_codes, sem_c0, sem_c1):
    wid = lax.axis_index("s") * _NC + lax.axis_index("c")
    base = wid * _PW

    for c in range(_NCHUNK):
        pltpu.sync_copy(pred_hbm.at[pl.ds(base + c * _CH, _CH)], idx_v.at[c])

    codes_cp = pltpu.async_copy(codes_hbm.at[:, pl.ds(base, _PW)], slab_v,
                                sem_codes)

    cent_bufs = (cents0, cents1)
    sems = (sem_c0, sem_c1)
    cps = [None, None]
    cps[0] = pltpu.async_copy(cents_hbm.at[idx_v.at[0]], cents0, sem_c0)

    codes_cp.wait()

    iota = lax.iota(jnp.int32, _L)
    lane_last = iota == (_L - 1)

    def t_group(g, _):
        dst0 = (g * _L + iota) * _CP
        for j in range(CODE_DIM):
            vals = slab_v[j, pl.ds(g * _L, _L)]
            plsc.store_scatter(codep_v, [dst0 + j], vals)
        return 0

    lax.fori_loop(0, _PW // _L, t_group, 0)

    for c in range(_NCHUNK):
        if c + 1 < _NCHUNK:
            nb = (c + 1) % 2
            cps[nb] = pltpu.async_copy(cents_hbm.at[idx_v.at[c + 1]],
                                       cent_bufs[nb], sems[nb])
        cps[c % 2].wait()
        cbuf = cent_bufs[c % 2]

        @plsc.parallel_loop(0, _CH, 1, unroll=4)
        def _item(i, c=c, cbuf=cbuf):
            row = c * _CH + i
            cbase = row * _CP + iota
            code = [plsc.load_gather(codep_v, [cbase + v * _L])
                    for v in range(_NV)]
            s = []
            for k in range(K):
                acc = jnp.abs(code[0] - cbuf[i, pl.ds(k * CODE_DIM, _L)])
                for v in range(1, _NV):
                    t = cbuf[i, pl.ds(k * CODE_DIM + v * _L, _L)]
                    acc = acc + jnp.abs(code[v] - t)
                s.append(plsc.cumsum(acc))
            m = jnp.minimum(jnp.minimum(s[0], s[1]), jnp.minimum(s[2], s[3]))
            m = m * (1.0 / CODE_DIM)
            pos = jnp.full((_L,), row, jnp.int32)
            plsc.store_scatter(out_v, [pos], m, mask=lane_last)

    pltpu.sync_copy(out_v, out_hbm.at[pl.ds(base, _PW)])


_mesh = plsc.VectorSubcoreMesh(core_axis_name="c", subcore_axis_name="s")

_sc_kernel = pl.kernel(
    _sc_body,
    mesh=_mesh,
    out_type=jax.ShapeDtypeStruct((BATCH,), jnp.float32),
    scratch_types=[
        pltpu.VMEM((_NCHUNK, _CH), jnp.int32),
        pltpu.VMEM((CODE_DIM, _PW), jnp.float32),
        pltpu.VMEM((_PW * _CP,), jnp.float32),
        pltpu.VMEM((_CH, _ROWD), jnp.float32),
        pltpu.VMEM((_CH, _ROWD), jnp.float32),
        pltpu.VMEM((_PW,), jnp.float32),
        pltpu.SemaphoreType.DMA,
        pltpu.SemaphoreType.DMA,
        pltpu.SemaphoreType.DMA,
    ],
    compiler_params=pltpu.CompilerParams(needs_layout_passes=False),
)


def kernel(codes, pred_class, centroids):
    pred = pred_class.astype(jnp.int32)
    cents = centroids.reshape(NUM_CLASSES, _ROWD)
    return _sc_kernel(codes.T, pred, cents)


# speedup vs baseline: 2.3348x; 1.0232x over previous
import jax
import jax.numpy as jnp
from jax import lax
from jax.experimental import pallas as pl
from jax.experimental.pallas import tpu as pltpu
from jax.experimental.pallas import tpu_sc as plsc

NUM_CLASSES = 100000
CODE_DIM = 64
K = 4
BATCH = 16384

_info = plsc.get_sparse_core_info()
_NC, _NS, _L = _info.num_cores, _info.num_subcores, _info.num_lanes
_NW = _NC * _NS
_PW = BATCH // _NW
_CH = 128
_NCHUNK = _PW // _CH
_NV = CODE_DIM // _L
_ROWD = K * CODE_DIM
_CP = CODE_DIM + 1


def _sc_body(codes_hbm, pred_hbm, cents_hbm, out_hbm,
             idx_v, slab0, slab1, codep_v, cents0, cents1, out_v,
             sem_codes0, sem_codes1, sem_c0, sem_c1):
    wid = lax.axis_index("s") * _NC + lax.axis_index("c")
    base = wid * _PW

    for c in range(_NCHUNK):
        pltpu.sync_copy(pred_hbm.at[pl.ds(base + c * _CH, _CH)], idx_v.at[c])

    cent_bufs = (cents0, cents1)
    sems = (sem_c0, sem_c1)
    cps = [None, None]
    cps[0] = pltpu.async_copy(cents_hbm.at[idx_v.at[0]], cents0, sem_c0)
    cps[1] = pltpu.async_copy(cents_hbm.at[idx_v.at[1]], cents1, sem_c1)

    iota = lax.iota(jnp.int32, _L)
    lane_last = iota == (_L - 1)

    slab_bufs = (slab0, slab1)
    csems = (sem_codes0, sem_codes1)
    scps = [None, None]
    scps[0] = pltpu.async_copy(codes_hbm.at[:, pl.ds(base, _CH)], slab0,
                               sem_codes0)
    for c in range(_NCHUNK):
        if c + 1 < _NCHUNK:
            nb = (c + 1) % 2
            scps[nb] = pltpu.async_copy(
                codes_hbm.at[:, pl.ds(base + (c + 1) * _CH, _CH)],
                slab_bufs[nb], csems[nb])
        scps[c % 2].wait()
        sbuf = slab_bufs[c % 2]

        def t_group(g, _, c=c, sbuf=sbuf):
            dst0 = (c * _CH + g * _L + iota) * _CP
            for j in range(CODE_DIM):
                vals = sbuf[j, pl.ds(g * _L, _L)]
                plsc.store_scatter(codep_v, [dst0 + j], vals)
            return 0

        lax.fori_loop(0, _CH // _L, t_group, 0)

    for c in range(_NCHUNK):
        cps[c % 2].wait()
        cbuf = cent_bufs[c % 2]

        @plsc.parallel_loop(0, _CH, 1, unroll=4)
        def _item(i, c=c, cbuf=cbuf):
            row = c * _CH + i
            cbase = row * _CP + iota
            code = [plsc.load_gather(codep_v, [cbase + v * _L])
                    for v in range(_NV)]
            s = []
            for k in range(K):
                acc = jnp.abs(code[0] - cbuf[i, pl.ds(k * CODE_DIM, _L)])
                for v in range(1, _NV):
                    t = cbuf[i, pl.ds(k * CODE_DIM + v * _L, _L)]
                    acc = acc + jnp.abs(code[v] - t)
                s.append(plsc.cumsum(acc))
            m = jnp.minimum(jnp.minimum(s[0], s[1]), jnp.minimum(s[2], s[3]))
            m = m * (1.0 / CODE_DIM)
            pos = jnp.full((_L,), row, jnp.int32)
            plsc.store_scatter(out_v, [pos], m, mask=lane_last)

        if c + 2 < _NCHUNK:
            nb = c % 2
            cps[nb] = pltpu.async_copy(cents_hbm.at[idx_v.at[c + 2]],
                                       cent_bufs[nb], sems[nb])

    pltpu.sync_copy(out_v, out_hbm.at[pl.ds(base, _PW)])


_mesh = plsc.VectorSubcoreMesh(core_axis_name="c", subcore_axis_name="s")

_sc_kernel = pl.kernel(
    _sc_body,
    mesh=_mesh,
    out_type=jax.ShapeDtypeStruct((BATCH,), jnp.float32),
    scratch_types=[
        pltpu.VMEM((_NCHUNK, _CH), jnp.int32),
        pltpu.VMEM((CODE_DIM, _CH), jnp.float32),
        pltpu.VMEM((CODE_DIM, _CH), jnp.float32),
        pltpu.VMEM((_PW * _CP,), jnp.float32),
        pltpu.VMEM((_CH, _ROWD), jnp.float32),
        pltpu.VMEM((_CH, _ROWD), jnp.float32),
        pltpu.VMEM((_PW,), jnp.float32),
        pltpu.SemaphoreType.DMA,
        pltpu.SemaphoreType.DMA,
        pltpu.SemaphoreType.DMA,
        pltpu.SemaphoreType.DMA,
    ],
    compiler_params=pltpu.CompilerParams(needs_layout_passes=False),
)


def kernel(codes, pred_class, centroids):
    pred = pred_class.astype(jnp.int32)
    cents = centroids.reshape(NUM_CLASSES, _ROWD)
    return _sc_kernel(codes.T, pred, cents)
